# Initial kernel scaffold; baseline (speedup 1.0000x reference)
import jax
import jax.numpy as jnp
from jax.experimental import pallas as pl


def kernel(features, user_graph, user_matrix):
    raise NotImplementedError("write your pallas kernel here")


# speedup vs baseline: 1.7202x; 1.7202x over previous
import functools

import jax
import jax.numpy as jnp
from jax import lax
from jax.experimental import pallas as pl
from jax.experimental.pallas import tpu as pltpu
from jax.experimental.pallas import tpu_sc as plsc

NC = 2
NS = 16
L = 16
NW = NC * NS

U_STEP = 4


def _make_kernel(NP, K, D, n_feat):
    C = NP // NW
    n_steps = C // U_STEP
    E = U_STEP * K
    DV = D // L

    mesh = plsc.VectorSubcoreMesh(core_axis_name="c", subcore_axis_name="s")

    @functools.partial(
        pl.kernel,
        out_type=jax.ShapeDtypeStruct((NP, D), jnp.float32),
        mesh=mesh,
        scratch_types=[
            pltpu.VMEM((E,), jnp.int32),
            pltpu.VMEM((E,), jnp.float32),
            pltpu.VMEM((E, D), jnp.float32),
            pltpu.VMEM((U_STEP, D), jnp.float32),
            pltpu.SemaphoreType.DMA,
        ],
    )
    def kern(feat_hbm, gidx_hbm, w_hbm, out_hbm, idx_v, w_v, rows_v, out_v, sem):
        wid = lax.axis_index("s") * NC + lax.axis_index("c")
        base_u = wid * C

        def step(s, carry):
            u0 = base_u + s * U_STEP
            e0 = u0 * K
            pltpu.sync_copy(gidx_hbm.at[pl.ds(e0, E)], idx_v)
            pltpu.sync_copy(w_hbm.at[pl.ds(e0, E)], w_v)
            pltpu.async_copy(feat_hbm.at[idx_v], rows_v, sem).wait()

            def user(u, c):
                acc = [jnp.zeros((L,), jnp.float32) for _ in range(DV)]
                wv = [w_v[pl.ds(u * K + j * L, L)] for j in range(K // L)]
                for k in range(K):
                    e = u * K + k
                    w = wv[k // L][k % L]
                    for d in range(DV):
                        acc[d] = acc[d] + w * rows_v[e, pl.ds(d * L, L)]
                for d in range(DV):
                    out_v[u, pl.ds(d * L, L)] = acc[d]
                return c

            lax.fori_loop(0, U_STEP, user, 0)
            pltpu.sync_copy(out_v, out_hbm.at[pl.ds(u0, U_STEP), :])
            return carry

        lax.fori_loop(0, n_steps, step, 0)

    return kern


def kernel(features, user_graph, user_matrix):
    N, K = user_graph.shape
    n_feat, D = features.shape
    chunk = NW * U_STEP
    NP = ((N + chunk - 1) // chunk) * chunk

    gidx = jnp.reshape(user_graph.astype(jnp.int32), (N * K,))
    w = jnp.reshape(user_matrix.astype(jnp.float32), (N * K,))
    pad = NP * K - N * K
    if pad:
        gidx = jnp.pad(gidx, (0, pad))
        w = jnp.pad(w, (0, pad))

    out = _make_kernel(NP, K, D, n_feat)(features.astype(jnp.float32), gidx, w)
    return out[:N]
